# baseline (speedup 1.0000x reference)
import functools

import jax
import jax.numpy as jnp
from jax import lax
from jax.experimental import pallas as pl
from jax.experimental.pallas import tpu as pltpu

E = 8
TOPK = 2
T = 2048
D = 1024
F = 1024
A = T * TOPK
BM = 256
NB = A // BM
NT = NB + E - 1


def _routing_plan(top_k_index, top_k_weights):
    flat_e = top_k_index.reshape(A).astype(jnp.int32)
    onehot = (flat_e[:, None] == jnp.arange(E, dtype=jnp.int32)[None, :])
    csum = jnp.cumsum(onehot.astype(jnp.int32), axis=0)
    counts = csum[-1]
    offsets = jnp.concatenate(
        [jnp.zeros((1,), jnp.int32), jnp.cumsum(counts, dtype=jnp.int32)])
    rank = jnp.take_along_axis(csum, flat_e[:, None], axis=1)[:, 0] - 1
    pos = offsets[flat_e] + rank
    order = jnp.zeros((A,), jnp.int32).at[pos].set(
        jnp.arange(A, dtype=jnp.int32))
    src_row = order // TOPK
    w_sorted = jnp.take(top_k_weights.reshape(A), order)

    start, end = offsets[:E], offsets[1:]
    nonempty = end > start
    first_blk = start // BM
    nb_e = jnp.where(nonempty, (end - 1) // BM - first_blk + 1, 0)
    cum_t = jnp.concatenate(
        [jnp.zeros((1,), jnp.int32), jnp.cumsum(nb_e, dtype=jnp.int32)])
    total = cum_t[E]
    slot = jnp.arange(NT, dtype=jnp.int32)
    valid = slot < total
    e_of = jnp.clip(
        jnp.searchsorted(cum_t, slot, side="right").astype(jnp.int32) - 1,
        0, E - 1)
    m_of = jnp.where(valid, first_blk[e_of] + (slot - cum_t[e_of]), NB - 1)
    lo = jnp.where(valid, jnp.maximum(start[e_of], m_of * BM), 0)
    hi = jnp.where(valid, jnp.minimum(end[e_of], (m_of + 1) * BM), 0)
    e_t = jnp.where(valid, e_of, 0)
    prev_m = jnp.concatenate([jnp.full((1,), -1, jnp.int32), m_of[:-1]])
    first = (m_of != prev_m).astype(jnp.int32)
    return src_row, pos, w_sorted, m_of, e_t, lo, hi, first


def _ffn_tile(m_r, e_r, lo_r, hi_r, first_r, x_ref, wu_ref, wd_ref, w_ref,
              o_ref):
    i = pl.program_id(0)
    m, lo, hi, first = m_r[i], lo_r[i], hi_r[i], first_r[i]
    g = m * BM + lax.broadcasted_iota(jnp.int32, (BM, 1), 0)
    w = jnp.where((g >= lo) & (g < hi), w_ref[0], 0.0)
    h = jnp.dot(x_ref[...], wu_ref[0], preferred_element_type=jnp.float32)
    a = jnp.maximum(h, 0.0)
    o = jnp.dot(a * a, wd_ref[0], preferred_element_type=jnp.float32)
    contrib = w * o

    @pl.when(first == 1)
    def _():
        o_ref[...] = contrib

    @pl.when(first == 0)
    def _():
        o_ref[...] += contrib


def _grouped_ffn(x_sorted, w_up, w_down, w_s, tile_m, tile_e, tile_lo,
                 tile_hi, tile_first):
    grid_spec = pltpu.PrefetchScalarGridSpec(
        num_scalar_prefetch=5,
        grid=(NT,),
        in_specs=[
            pl.BlockSpec((BM, D), lambda i, m, e, lo, hi, f: (m[i], 0)),
            pl.BlockSpec((1, D, F), lambda i, m, e, lo, hi, f: (e[i], 0, 0)),
            pl.BlockSpec((1, F, D), lambda i, m, e, lo, hi, f: (e[i], 0, 0)),
            pl.BlockSpec((1, BM, 1), lambda i, m, e, lo, hi, f: (m[i], 0, 0)),
        ],
        out_specs=pl.BlockSpec((BM, D), lambda i, m, e, lo, hi, f: (m[i], 0)),
    )
    return pl.pallas_call(
        _ffn_tile,
        grid_spec=grid_spec,
        out_shape=jax.ShapeDtypeStruct((A, D), jnp.float32),
        compiler_params=pltpu.CompilerParams(
            dimension_semantics=("arbitrary",)),
    )(tile_m, tile_e, tile_lo, tile_hi, tile_first,
      x_sorted, w_up, w_down, w_s)


def kernel(hidden_states, top_k_index, top_k_weights, W_up, W_down):
    (src_row, pos, w_sorted, tile_m, tile_e, tile_lo, tile_hi,
     tile_first) = _routing_plan(top_k_index, top_k_weights)
    x_sorted = jnp.take(hidden_states, src_row, axis=0)
    w_s = w_sorted.reshape(NB, BM, 1)
    o_sorted = _grouped_ffn(x_sorted, W_up, W_down, w_s, tile_m, tile_e,
                            tile_lo, tile_hi, tile_first)
    p = pos.reshape(T, TOPK)
    return jnp.take(o_sorted, p[:, 0], axis=0) + jnp.take(
        o_sorted, p[:, 1], axis=0)
